# Initial kernel scaffold; baseline (speedup 1.0000x reference)
import jax
import jax.numpy as jnp
from jax.experimental import pallas as pl


def kernel(h, edge_index, W1, b1, W2, b2):
    raise NotImplementedError("write your pallas kernel here")


# speedup vs baseline: 3.3320x; 3.3320x over previous
import functools

import jax
import jax.numpy as jnp
from jax import lax
from jax.experimental import pallas as pl
from jax.experimental.pallas import tpu as pltpu
from jax.experimental.pallas import tpu_sc as plsc

N = 10000
E = 320000
F = 128
NC = 2
NS = 16
NW = NC * NS
EW = E // NW
CH = 64
T = 160
EP = T * CH
DUMMY = N
NX = 10016
NPAD = 10240
DS = 16
BK = 16

_mesh = plsc.VectorSubcoreMesh(
    core_axis_name="c", subcore_axis_name="s", num_cores=NC, num_subcores=NS
)


def _zero_rows(buf, rows):
    z = jnp.zeros((16,), jnp.float32)
    w = buf.shape[1]
    for r in range(rows):
        for c in range(w // 16):
            buf[r, pl.ds(c * 16, 16)] = z


def _deg_body(src_hbm, dst_hbm, dego_hbm, degi_hbm,
              sidx, didx, ones_v, obuf, acco, acci, semz):
    c = lax.axis_index("c")
    s = lax.axis_index("s")
    wid = c * NS + s
    one = jnp.ones((16,), jnp.float32)
    zero = jnp.zeros((16,), jnp.float32)
    for r in range(CH // 16):
        ones_v[pl.ds(r * 16, 16)] = one
    for r in range(8):
        obuf[pl.ds(r * 16, 16)] = zero

    @pl.loop(0, NPAD // NS // 128)
    def _zero(i):
        base = s * (NPAD // NS) + i * 128
        pltpu.sync_copy(obuf, acco.at[pl.ds(base, 128)])
        pltpu.sync_copy(obuf, acci.at[pl.ds(base, 128)])

    plsc.subcore_barrier()

    @pl.loop(0, T // BK)
    def _scat(o):
        pltpu.sync_copy(src_hbm.at[wid, pl.ds(o * BK, BK)], sidx)
        pltpu.sync_copy(dst_hbm.at[wid, pl.ds(o * BK, BK)], didx)
        for k in range(BK):
            pltpu.sync_copy(ones_v, acco.at[sidx.at[k]], add=True)
            pltpu.sync_copy(ones_v, acci.at[didx.at[k]], add=True)

    plsc.subcore_barrier()

    @pl.loop(0, 5)
    def _out(i):
        base = s * (NPAD // NS) + i * 128
        pltpu.sync_copy(acco.at[pl.ds(base, 128)], obuf)
        pltpu.sync_copy(obuf, dego_hbm.at[c, pl.ds(base, 128)])
        pltpu.sync_copy(acci.at[pl.ds(base, 128)], obuf)
        pltpu.sync_copy(obuf, degi_hbm.at[c, pl.ds(base, 128)])


_deg_kernel = pl.kernel(
    _deg_body,
    out_type=(
        jax.ShapeDtypeStruct((NC, NPAD), jnp.float32),
        jax.ShapeDtypeStruct((NC, NPAD), jnp.float32),
    ),
    mesh=_mesh,
    scratch_types=[
        pltpu.VMEM((BK, CH), jnp.int32),
        pltpu.VMEM((BK, CH), jnp.int32),
        pltpu.VMEM((CH,), jnp.float32),
        pltpu.VMEM((128,), jnp.float32),
        pltpu.VMEM_SHARED((NPAD,), jnp.float32),
        pltpu.VMEM_SHARED((NPAD,), jnp.float32),
        pltpu.SemaphoreType.DMA,
    ],
)


def _prop_body(x_hbm, src_hbm, dst_hbm, out_hbm,
               sidx, didx, buf0, buf1, acc, gsem):
    c = lax.axis_index("c")
    s = lax.axis_index("s")
    wid = c * NS + s
    _zero_rows(buf0, 16)

    @pl.loop(0, NPAD // NS // 16)
    def _zero(i):
        base = s * (NPAD // NS) + i * 16
        pltpu.sync_copy(buf0.at[pl.ds(0, 16)], acc.at[pl.ds(base, 16)])

    plsc.subcore_barrier()

    @pl.loop(0, T // BK)
    def _blk(o):
        pltpu.sync_copy(src_hbm.at[wid, pl.ds(o * BK, BK)], sidx)
        pltpu.sync_copy(dst_hbm.at[wid, pl.ds(o * BK, BK)], didx)
        pltpu.async_copy(x_hbm.at[sidx.at[0]], buf0, gsem).wait()
        for k in range(0, BK, 2):
            cp1 = pltpu.async_copy(x_hbm.at[sidx.at[k + 1]], buf1, gsem)
            pltpu.sync_copy(buf0, acc.at[didx.at[k]], add=True)
            cp1.wait()
            if k + 2 < BK:
                cp2 = pltpu.async_copy(x_hbm.at[sidx.at[k + 2]], buf0, gsem)
                pltpu.sync_copy(buf1, acc.at[didx.at[k + 1]], add=True)
                cp2.wait()
            else:
                pltpu.sync_copy(buf1, acc.at[didx.at[k + 1]], add=True)

    plsc.subcore_barrier()

    @pl.loop(0, NPAD // NS // CH)
    def _out(i):
        base = s * (NPAD // NS) + i * CH
        pltpu.sync_copy(acc.at[pl.ds(base, CH)], buf0)
        pltpu.sync_copy(buf0, out_hbm.at[c].at[pl.ds(base, CH)])


_prop_kernel = pl.kernel(
    _prop_body,
    out_type=jax.ShapeDtypeStruct((NC, NPAD, F), jnp.float32),
    mesh=_mesh,
    scratch_types=[
        pltpu.VMEM((BK, CH), jnp.int32),
        pltpu.VMEM((BK, CH), jnp.int32),
        pltpu.VMEM((CH, F), jnp.float32),
        pltpu.VMEM((CH, F), jnp.float32),
        pltpu.VMEM_SHARED((NPAD, F), jnp.float32),
        pltpu.SemaphoreType.DMA,
    ],
)


def _rsqrt_deg(deg_ref):
    d = deg_ref[...]
    cnt = d[0, :N] + d[1, :N]
    return lax.rsqrt(jnp.maximum(cnt, 1.0))[:, None]


def _scale_body(x_ref, dego_ref, o_ref):
    xs = x_ref[...] * _rsqrt_deg(dego_ref)
    o_ref[pl.ds(0, N), :] = xs
    o_ref[pl.ds(N, NX - N), :] = jnp.zeros((NX - N, F), jnp.float32)


def _mid_body(yp_ref, degi_ref, dego_ref, w_ref, b_ref, o_ref):
    yp = yp_ref[...]
    y = (yp[0, :N] + yp[1, :N]) * _rsqrt_deg(degi_ref)
    z = jnp.dot(y, w_ref[...], preferred_element_type=jnp.float32) + b_ref[...]
    z = jnp.maximum(z, 0.0) * _rsqrt_deg(dego_ref)
    o_ref[pl.ds(0, N), :] = z
    o_ref[pl.ds(N, NX - N), :] = jnp.zeros((NX - N, F), jnp.float32)


def _fin_body(yp_ref, degi_ref, w_ref, b_ref, o_ref):
    yp = yp_ref[...]
    y = (yp[0, :N] + yp[1, :N]) * _rsqrt_deg(degi_ref)
    o_ref[...] = (
        jnp.dot(y, w_ref[...], preferred_element_type=jnp.float32) + b_ref[...]
    )


def _pad_edges(idx):
    w = idx.reshape(NW, EW)
    pad = jnp.full((NW, EP - EW), DUMMY, jnp.int32)
    return jnp.concatenate([w, pad], axis=1).reshape(NW, T, CH)


def kernel(h, edge_index, W1, b1, W2, b2):
    srcp = _pad_edges(edge_index[0])
    dstp = _pad_edges(edge_index[1])
    dego, degi = _deg_kernel(srcp, dstp)

    x0 = h.T
    xs = pl.pallas_call(
        _scale_body, out_shape=jax.ShapeDtypeStruct((NX, F), jnp.float32)
    )(x0, dego)
    y1 = _prop_kernel(xs, srcp, dstp)
    z1 = pl.pallas_call(
        _mid_body, out_shape=jax.ShapeDtypeStruct((NX, F), jnp.float32)
    )(y1, degi, dego, W1, b1.reshape(1, F))
    y2 = _prop_kernel(z1, srcp, dstp)
    out = pl.pallas_call(
        _fin_body, out_shape=jax.ShapeDtypeStruct((N, F), jnp.float32)
    )(y2, degi, W2, b2.reshape(1, F))
    return out.T


# speedup vs baseline: 3.6733x; 1.1024x over previous
import functools

import jax
import jax.numpy as jnp
from jax import lax
from jax.experimental import pallas as pl
from jax.experimental.pallas import tpu as pltpu
from jax.experimental.pallas import tpu_sc as plsc

N = 10000
E = 320000
F = 128
NC = 2
NS = 16
NW = NC * NS
EW = E // NW
CH = 128
T = 80
EP = T * CH
DUMMY = N
NX = 10016
NPAD = 10240
DS = 16
BK = 8

_mesh = plsc.VectorSubcoreMesh(
    core_axis_name="c", subcore_axis_name="s", num_cores=NC, num_subcores=NS
)


def _zero_rows(buf, rows):
    z = jnp.zeros((16,), jnp.float32)
    w = buf.shape[1]
    for r in range(rows):
        for c in range(w // 16):
            buf[r, pl.ds(c * 16, 16)] = z


def _deg_body(src_hbm, dst_hbm, dego_hbm, degi_hbm,
              sidx, didx, ones_v, obuf, acco, acci, semz):
    c = lax.axis_index("c")
    s = lax.axis_index("s")
    wid = c * NS + s
    one = jnp.ones((16,), jnp.float32)
    zero = jnp.zeros((16,), jnp.float32)
    for r in range(CH // 16):
        ones_v[pl.ds(r * 16, 16)] = one
    for r in range(8):
        obuf[pl.ds(r * 16, 16)] = zero

    @pl.loop(0, NPAD // NS // 128)
    def _zero(i):
        base = s * (NPAD // NS) + i * 128
        pltpu.sync_copy(obuf, acco.at[pl.ds(base, 128)])
        pltpu.sync_copy(obuf, acci.at[pl.ds(base, 128)])

    plsc.subcore_barrier()

    @pl.loop(0, T // BK)
    def _scat(o):
        pltpu.sync_copy(src_hbm.at[wid, pl.ds(o * BK, BK)], sidx)
        pltpu.sync_copy(dst_hbm.at[wid, pl.ds(o * BK, BK)], didx)
        for k in range(BK):
            pltpu.sync_copy(ones_v, acco.at[sidx.at[k]], add=True)
            pltpu.sync_copy(ones_v, acci.at[didx.at[k]], add=True)

    plsc.subcore_barrier()

    @pl.loop(0, 5)
    def _out(i):
        base = s * (NPAD // NS) + i * 128
        pltpu.sync_copy(acco.at[pl.ds(base, 128)], obuf)
        pltpu.sync_copy(obuf, dego_hbm.at[c, pl.ds(base, 128)])
        pltpu.sync_copy(acci.at[pl.ds(base, 128)], obuf)
        pltpu.sync_copy(obuf, degi_hbm.at[c, pl.ds(base, 128)])


_deg_kernel = pl.kernel(
    _deg_body,
    out_type=(
        jax.ShapeDtypeStruct((NC, NPAD), jnp.float32),
        jax.ShapeDtypeStruct((NC, NPAD), jnp.float32),
    ),
    mesh=_mesh,
    scratch_types=[
        pltpu.VMEM((BK, CH), jnp.int32),
        pltpu.VMEM((BK, CH), jnp.int32),
        pltpu.VMEM((CH,), jnp.float32),
        pltpu.VMEM((128,), jnp.float32),
        pltpu.VMEM_SHARED((NPAD,), jnp.float32),
        pltpu.VMEM_SHARED((NPAD,), jnp.float32),
        pltpu.SemaphoreType.DMA,
    ],
)


def _prop_body(x_hbm, src_hbm, dst_hbm, out_hbm,
               sidx, didx, buf0, buf1, acc, gsem):
    c = lax.axis_index("c")
    s = lax.axis_index("s")
    wid = c * NS + s
    _zero_rows(buf0, 16)

    @pl.loop(0, NPAD // NS // 16)
    def _zero(i):
        base = s * (NPAD // NS) + i * 16
        pltpu.sync_copy(buf0.at[pl.ds(0, 16)], acc.at[pl.ds(base, 16)])

    plsc.subcore_barrier()

    @pl.loop(0, T // BK)
    def _blk(o):
        pltpu.sync_copy(src_hbm.at[wid, pl.ds(o * BK, BK)], sidx)
        pltpu.sync_copy(dst_hbm.at[wid, pl.ds(o * BK, BK)], didx)
        pltpu.async_copy(x_hbm.at[sidx.at[0]], buf0, gsem).wait()
        for k in range(0, BK, 2):
            cp1 = pltpu.async_copy(x_hbm.at[sidx.at[k + 1]], buf1, gsem)
            pltpu.sync_copy(buf0, acc.at[didx.at[k]], add=True)
            cp1.wait()
            if k + 2 < BK:
                cp2 = pltpu.async_copy(x_hbm.at[sidx.at[k + 2]], buf0, gsem)
                pltpu.sync_copy(buf1, acc.at[didx.at[k + 1]], add=True)
                cp2.wait()
            else:
                pltpu.sync_copy(buf1, acc.at[didx.at[k + 1]], add=True)

    plsc.subcore_barrier()

    @pl.loop(0, NPAD // NS // CH)
    def _out(i):
        base = s * (NPAD // NS) + i * CH
        pltpu.sync_copy(acc.at[pl.ds(base, CH)], buf0)
        pltpu.sync_copy(buf0, out_hbm.at[c].at[pl.ds(base, CH)])


_prop_kernel = pl.kernel(
    _prop_body,
    out_type=jax.ShapeDtypeStruct((NC, NPAD, F), jnp.float32),
    mesh=_mesh,
    scratch_types=[
        pltpu.VMEM((BK, CH), jnp.int32),
        pltpu.VMEM((BK, CH), jnp.int32),
        pltpu.VMEM((CH, F), jnp.float32),
        pltpu.VMEM((CH, F), jnp.float32),
        pltpu.VMEM_SHARED((NPAD, F), jnp.float32),
        pltpu.SemaphoreType.DMA,
    ],
)


def _rsqrt_deg(deg_ref):
    d = deg_ref[...]
    cnt = d[0, :N] + d[1, :N]
    return lax.rsqrt(jnp.maximum(cnt, 1.0))[:, None]


def _scale_body(x_ref, dego_ref, o_ref):
    xs = x_ref[...] * _rsqrt_deg(dego_ref)
    o_ref[pl.ds(0, N), :] = xs
    o_ref[pl.ds(N, NX - N), :] = jnp.zeros((NX - N, F), jnp.float32)


def _mid_body(yp_ref, degi_ref, dego_ref, w_ref, b_ref, o_ref):
    yp = yp_ref[...]
    y = (yp[0, :N] + yp[1, :N]) * _rsqrt_deg(degi_ref)
    z = jnp.dot(y, w_ref[...], preferred_element_type=jnp.float32) + b_ref[...]
    z = jnp.maximum(z, 0.0) * _rsqrt_deg(dego_ref)
    o_ref[pl.ds(0, N), :] = z
    o_ref[pl.ds(N, NX - N), :] = jnp.zeros((NX - N, F), jnp.float32)


def _fin_body(yp_ref, degi_ref, w_ref, b_ref, o_ref):
    yp = yp_ref[...]
    y = (yp[0, :N] + yp[1, :N]) * _rsqrt_deg(degi_ref)
    o_ref[...] = (
        jnp.dot(y, w_ref[...], preferred_element_type=jnp.float32) + b_ref[...]
    )


def _pad_edges(idx):
    w = idx.reshape(NW, EW)
    pad = jnp.full((NW, EP - EW), DUMMY, jnp.int32)
    return jnp.concatenate([w, pad], axis=1).reshape(NW, T, CH)


def kernel(h, edge_index, W1, b1, W2, b2):
    srcp = _pad_edges(edge_index[0])
    dstp = _pad_edges(edge_index[1])
    dego, degi = _deg_kernel(srcp, dstp)

    x0 = h.T
    xs = pl.pallas_call(
        _scale_body, out_shape=jax.ShapeDtypeStruct((NX, F), jnp.float32)
    )(x0, dego)
    y1 = _prop_kernel(xs, srcp, dstp)
    z1 = pl.pallas_call(
        _mid_body, out_shape=jax.ShapeDtypeStruct((NX, F), jnp.float32)
    )(y1, degi, dego, W1, b1.reshape(1, F))
    y2 = _prop_kernel(z1, srcp, dstp)
    out = pl.pallas_call(
        _fin_body, out_shape=jax.ShapeDtypeStruct((N, F), jnp.float32)
    )(y2, degi, W2, b2.reshape(1, F))
    return out.T


# speedup vs baseline: 3.6765x; 1.0009x over previous
import functools

import jax
import jax.numpy as jnp
from jax import lax
from jax.experimental import pallas as pl
from jax.experimental.pallas import tpu as pltpu
from jax.experimental.pallas import tpu_sc as plsc

N = 10000
E = 320000
F = 128
NC = 2
NS = 16
NW = NC * NS
EW = E // NW
CH = 128
T = 80
EP = T * CH
DUMMY = N
NX = 10016
NPAD = 10240
DS = 16
BK = 8

_mesh = plsc.VectorSubcoreMesh(
    core_axis_name="c", subcore_axis_name="s", num_cores=NC, num_subcores=NS
)


def _zero_rows(buf, rows):
    z = jnp.zeros((16,), jnp.float32)
    w = buf.shape[1]
    for r in range(rows):
        for c in range(w // 16):
            buf[r, pl.ds(c * 16, 16)] = z


def _deg_body(src_hbm, dst_hbm, dego_hbm, degi_hbm,
              sidx, didx, ones_v, obuf, acco, acci, semz):
    c = lax.axis_index("c")
    s = lax.axis_index("s")
    wid = c * NS + s
    one = jnp.ones((16,), jnp.float32)
    zero = jnp.zeros((16,), jnp.float32)
    for r in range(CH // 16):
        ones_v[pl.ds(r * 16, 16)] = one
    for r in range(8):
        obuf[pl.ds(r * 16, 16)] = zero

    @pl.loop(0, NPAD // NS // 128)
    def _zero(i):
        base = s * (NPAD // NS) + i * 128
        pltpu.sync_copy(obuf, acco.at[pl.ds(base, 128)])
        pltpu.sync_copy(obuf, acci.at[pl.ds(base, 128)])

    plsc.subcore_barrier()

    @pl.loop(0, T // BK)
    def _scat(o):
        pltpu.sync_copy(src_hbm.at[wid, pl.ds(o * BK, BK)], sidx)
        pltpu.sync_copy(dst_hbm.at[wid, pl.ds(o * BK, BK)], didx)
        for k in range(BK):
            pltpu.sync_copy(ones_v, acco.at[sidx.at[k]], add=True)
            pltpu.sync_copy(ones_v, acci.at[didx.at[k]], add=True)

    plsc.subcore_barrier()

    @pl.loop(0, 5)
    def _out(i):
        base = s * (NPAD // NS) + i * 128
        pltpu.sync_copy(acco.at[pl.ds(base, 128)], obuf)
        pltpu.sync_copy(obuf, dego_hbm.at[c, pl.ds(base, 128)])
        pltpu.sync_copy(acci.at[pl.ds(base, 128)], obuf)
        pltpu.sync_copy(obuf, degi_hbm.at[c, pl.ds(base, 128)])


_deg_kernel = pl.kernel(
    _deg_body,
    out_type=(
        jax.ShapeDtypeStruct((NC, NPAD), jnp.float32),
        jax.ShapeDtypeStruct((NC, NPAD), jnp.float32),
    ),
    mesh=_mesh,
    scratch_types=[
        pltpu.VMEM((BK, CH), jnp.int32),
        pltpu.VMEM((BK, CH), jnp.int32),
        pltpu.VMEM((CH,), jnp.float32),
        pltpu.VMEM((128,), jnp.float32),
        pltpu.VMEM_SHARED((NPAD,), jnp.float32),
        pltpu.VMEM_SHARED((NPAD,), jnp.float32),
        pltpu.SemaphoreType.DMA,
    ],
)


def _prop_body(x_hbm, src_hbm, dst_hbm, out_hbm,
               sidx, didx, buf0, buf1, acc, gsem):
    c = lax.axis_index("c")
    s = lax.axis_index("s")
    wid = c * NS + s
    _zero_rows(buf0, 16)

    @pl.loop(0, NPAD // NS // 16)
    def _zero(i):
        base = s * (NPAD // NS) + i * 16
        pltpu.sync_copy(buf0.at[pl.ds(0, 16)], acc.at[pl.ds(base, 16)])

    plsc.subcore_barrier()

    @pl.loop(0, T // BK)
    def _blk(o):
        pltpu.sync_copy(src_hbm.at[wid, pl.ds(o * BK, BK)], sidx)
        pltpu.sync_copy(dst_hbm.at[wid, pl.ds(o * BK, BK)], didx)
        pltpu.async_copy(x_hbm.at[sidx.at[0]], buf0, gsem).wait()
        for k in range(0, BK, 2):
            cp1 = pltpu.async_copy(x_hbm.at[sidx.at[k + 1]], buf1, gsem)
            pltpu.sync_copy(buf0, acc.at[didx.at[k]], add=True)
            cp1.wait()
            if k + 2 < BK:
                cp2 = pltpu.async_copy(x_hbm.at[sidx.at[k + 2]], buf0, gsem)
                pltpu.sync_copy(buf1, acc.at[didx.at[k + 1]], add=True)
                cp2.wait()
            else:
                pltpu.sync_copy(buf1, acc.at[didx.at[k + 1]], add=True)

    plsc.subcore_barrier()

    @pl.loop(0, NPAD // NS // CH)
    def _out(i):
        base = s * (NPAD // NS) + i * CH
        pltpu.sync_copy(acc.at[pl.ds(base, CH)], out_hbm.at[c, pl.ds(base, CH)])


_prop_kernel = pl.kernel(
    _prop_body,
    out_type=jax.ShapeDtypeStruct((NC, NPAD, F), jnp.float32),
    mesh=_mesh,
    scratch_types=[
        pltpu.VMEM((BK, CH), jnp.int32),
        pltpu.VMEM((BK, CH), jnp.int32),
        pltpu.VMEM((CH, F), jnp.float32),
        pltpu.VMEM((CH, F), jnp.float32),
        pltpu.VMEM_SHARED((NPAD, F), jnp.float32),
        pltpu.SemaphoreType.DMA,
    ],
)


def _rsqrt_deg(deg_ref):
    d = deg_ref[...]
    cnt = d[0, :N] + d[1, :N]
    return lax.rsqrt(jnp.maximum(cnt, 1.0))[:, None]


def _scale_body(x_ref, dego_ref, o_ref):
    xs = x_ref[...] * _rsqrt_deg(dego_ref)
    o_ref[pl.ds(0, N), :] = xs
    o_ref[pl.ds(N, NX - N), :] = jnp.zeros((NX - N, F), jnp.float32)


def _mid_body(yp_ref, degi_ref, dego_ref, w_ref, b_ref, o_ref):
    yp = yp_ref[...]
    y = (yp[0, :N] + yp[1, :N]) * _rsqrt_deg(degi_ref)
    z = jnp.dot(y, w_ref[...], preferred_element_type=jnp.float32) + b_ref[...]
    z = jnp.maximum(z, 0.0) * _rsqrt_deg(dego_ref)
    o_ref[pl.ds(0, N), :] = z
    o_ref[pl.ds(N, NX - N), :] = jnp.zeros((NX - N, F), jnp.float32)


def _fin_body(yp_ref, degi_ref, w_ref, b_ref, o_ref):
    yp = yp_ref[...]
    y = (yp[0, :N] + yp[1, :N]) * _rsqrt_deg(degi_ref)
    o_ref[...] = (
        jnp.dot(y, w_ref[...], preferred_element_type=jnp.float32) + b_ref[...]
    )


def _pad_edges(idx):
    w = idx.reshape(NW, EW)
    pad = jnp.full((NW, EP - EW), DUMMY, jnp.int32)
    return jnp.concatenate([w, pad], axis=1).reshape(NW, T, CH)


def kernel(h, edge_index, W1, b1, W2, b2):
    srcp = _pad_edges(edge_index[0])
    dstp = _pad_edges(edge_index[1])
    dego, degi = _deg_kernel(srcp, dstp)

    x0 = h.T
    xs = pl.pallas_call(
        _scale_body, out_shape=jax.ShapeDtypeStruct((NX, F), jnp.float32)
    )(x0, dego)
    y1 = _prop_kernel(xs, srcp, dstp)
    z1 = pl.pallas_call(
        _mid_body, out_shape=jax.ShapeDtypeStruct((NX, F), jnp.float32)
    )(y1, degi, dego, W1, b1.reshape(1, F))
    y2 = _prop_kernel(z1, srcp, dstp)
    out = pl.pallas_call(
        _fin_body, out_shape=jax.ShapeDtypeStruct((N, F), jnp.float32)
    )(y2, degi, W2, b2.reshape(1, F))
    return out.T
